# baseline (speedup 1.0000x reference)
import functools

import numpy as np
import jax
import jax.numpy as jnp
from jax import lax
from jax.experimental import pallas as pl
from jax.experimental.pallas import tpu as pltpu
from jax.experimental.pallas import tpu_sc as plsc

H_FFT = 384
W_FFT = 193
NUM_BANDS = 6
HIDDEN = 128
F = H_FFT * W_FFT
ROWS = 4 * 96
NC, NS = 2, 16
NW = NC * NS
ROWS_PER_W = ROWS // NW
PIECES = 8
PW = F // PIECES
CHUNKS = PW // 16


def _band_tables():
    yy = np.arange(H_FFT, dtype=np.float32).reshape(-1, 1)
    xx = np.arange(W_FFT, dtype=np.float32).reshape(1, -1)
    ry = yy / max(H_FFT - 1, 1)
    rx = xx / max(W_FFT - 1, 1)
    r = np.sqrt(ry ** 2 + rx ** 2)
    r = r / (r.max() + 1e-8)
    band = np.minimum(np.floor(r * NUM_BANDS), NUM_BANDS - 1)
    band = band.astype(np.int32).reshape(-1)
    counts = np.zeros(NUM_BANDS, dtype=np.float32)
    for b in range(NUM_BANDS):
        counts[b] = max(float((band == b).sum()), 1.0)
    inv = np.float32(1.0) / (counts + np.float32(1e-6))
    sidx = band * 16 + (np.arange(F, dtype=np.int32) % 16)
    return sidx.astype(np.int32), [float(v) for v in inv]


_SIDX_NP, _INV_COUNTS = _band_tables()

_MESH = plsc.VectorSubcoreMesh(core_axis_name="c", subcore_axis_name="s")


@functools.partial(
    pl.kernel,
    out_type=jax.ShapeDtypeStruct((ROWS, F), jnp.float32),
    mesh=_MESH,
    compiler_params=pltpu.CompilerParams(
        use_tc_tiling_on_sc=False, needs_layout_passes=False),
    scratch_types=[
        pltpu.VMEM((F,), jnp.int32),
        pltpu.VMEM((PW,), jnp.float32),
        pltpu.VMEM((PW,), jnp.float32),
        pltpu.VMEM((PW,), jnp.float32),
        pltpu.VMEM((PW,), jnp.float32),
        pltpu.VMEM((NUM_BANDS * 16,), jnp.float32),
        pltpu.VMEM((NUM_BANDS * 16,), jnp.float32),
        pltpu.VMEM((NUM_BANDS * HIDDEN,), jnp.float32),
        pltpu.VMEM((HIDDEN,), jnp.float32),
        pltpu.VMEM((HIDDEN * 16,), jnp.float32),
        pltpu.VMEM((16,), jnp.float32),
        pltpu.SemaphoreType.DMA,
        pltpu.SemaphoreType.DMA,
        pltpu.SemaphoreType.DMA,
        pltpu.SemaphoreType.DMA,
    ],
)
def _rbg(feat_hbm, sidx_hbm, w1_hbm, b1_hbm, w2_hbm, b2_hbm, out_hbm,
         sidx_v, fb0, fb1, ob0, ob1, acc96, gate96, w1v, b1v, w2v, b2v,
         semf0, semf1, semo0, semo1):
    wid = lax.axis_index("s") * NC + lax.axis_index("c")

    pltpu.sync_copy(sidx_hbm, sidx_v)
    pltpu.sync_copy(w1_hbm, w1v)
    pltpu.sync_copy(b1_hbm, b1v)
    pltpu.sync_copy(w2_hbm, w2v)
    pltpu.sync_copy(b2_hbm, b2v)

    zero16 = jnp.zeros((16,), jnp.float32)
    fbufs = (fb0, fb1)
    fsems = (semf0, semf1)
    obufs = (ob0, ob1)
    osems = (semo0, semo1)

    def row_body(r, carry):
        row = wid * ROWS_PER_W + r

        for k in range(NUM_BANDS):
            acc96[pl.ds(k * 16, 16)] = zero16
        handles = {}
        handles[0] = pltpu.async_copy(
            feat_hbm.at[row, pl.ds(0, PW)], fbufs[0], fsems[0])
        for p in range(PIECES):
            if p + 1 < PIECES:
                nb = (p + 1) % 2
                handles[p + 1] = pltpu.async_copy(
                    feat_hbm.at[row, pl.ds((p + 1) * PW, PW)],
                    fbufs[nb], fsems[nb])
            handles[p].wait()
            fb = fbufs[p % 2]

            def red_body(i, c, _p=p, _fb=fb):
                off = i * 16
                fv = _fb[pl.ds(off, 16)]
                iv = sidx_v[pl.ds(_p * PW + off, 16)]
                plsc.addupdate_scatter(acc96, [iv], fv)
                return c

            lax.fori_loop(0, CHUNKS, red_body, 0, unroll=4)

        def hsum(v):
            s = v[0]
            for l in range(1, 16):
                s = s + v[l]
            return s

        means = [hsum(acc96[pl.ds(k * 16, 16)]) * _INV_COUNTS[k]
                 for k in range(NUM_BANDS)]

        h_chunks = []
        for c8 in range(HIDDEN // 16):
            hv = b1v[pl.ds(c8 * 16, 16)]
            for k in range(NUM_BANDS):
                hv = hv + means[k] * w1v[pl.ds(k * HIDDEN + c8 * 16, 16)]
            h_chunks.append(jnp.maximum(hv, 0.0))

        av = b2v[...]
        for c8 in range(HIDDEN // 16):
            for l in range(16):
                j = c8 * 16 + l
                av = av + h_chunks[c8][l] * w2v[pl.ds(j * 16, 16)]
        av = 1.0 / (1.0 + jnp.exp(-av))
        for k in range(NUM_BANDS):
            gate96[pl.ds(k * 16, 16)] = zero16 + av[k]

        ohandles = {}
        for p in range(PIECES):
            ob = obufs[p % 2]
            if p >= 2:
                ohandles[p - 2].wait()

            def exp_body(i, c, _p=p, _ob=ob):
                off = i * 16
                iv = sidx_v[pl.ds(_p * PW + off, 16)]
                _ob[pl.ds(off, 16)] = plsc.load_gather(gate96, [iv])
                return c

            lax.fori_loop(0, CHUNKS, exp_body, 0, unroll=4)
            ohandles[p] = pltpu.async_copy(
                ob, out_hbm.at[row, pl.ds(p * PW, PW)], osems[p % 2])
        ohandles[PIECES - 2].wait()
        ohandles[PIECES - 1].wait()
        return carry

    lax.fori_loop(0, ROWS_PER_W, row_body, 0)


def kernel(feat_flat, W1, b1, W2, b2):
    B, C, Fdim = feat_flat.shape
    feat2 = feat_flat.reshape(B * C, Fdim)
    w2p = jnp.zeros((HIDDEN, 16), W2.dtype).at[:, :NUM_BANDS].set(W2)
    b2p = jnp.zeros((16,), b2.dtype).at[:NUM_BANDS].set(b2)
    out = _rbg(feat2, jnp.asarray(_SIDX_NP), W1.reshape(-1), b1,
               w2p.reshape(-1), b2p)
    return out.reshape(B, C, Fdim)
